# baseline (speedup 1.0000x reference)
import functools
import math

import jax
import jax.numpy as jnp
from jax import lax
from jax.experimental import pallas as pl
from jax.experimental.pallas import tpu as pltpu
from jax.experimental.pallas import tpu_sc as plsc

_NUM_EXPERTS = 8
_TOP_K = 2
_CAP_FACTOR = 1.25
_MIN_CAPACITY = 4


def _capacity(num_tokens, num_experts):
    cap = math.floor(_TOP_K * _CAP_FACTOR * num_tokens / num_experts)
    cap += cap % 2
    return max(cap, _MIN_CAPACITY)


def _router_body(cap, tok_ref, gate_ref, idx_ref, w1_ref, w2_ref, src_ref,
                 tokext_ref):
    T = tok_ref.shape[0]
    E = gate_ref.shape[0]
    x = tok_ref[...]
    g = gate_ref[...]
    logits = lax.dot_general(x, g, (((1,), (1,)), ((), ())),
                             preferred_element_type=jnp.float32)
    m = jnp.max(logits, axis=-1, keepdims=True)
    p = jnp.exp(logits - m)
    probs = p / jnp.sum(p, axis=-1, keepdims=True)

    ei = lax.broadcasted_iota(jnp.int32, (T, E), 1)
    p1 = jnp.max(probs, axis=-1, keepdims=True)
    e1 = jnp.min(jnp.where(probs >= p1, ei, E), axis=-1, keepdims=True)
    mask1 = ei == e1
    probsm = jnp.where(mask1, -1.0, probs)
    p2 = jnp.max(probsm, axis=-1, keepdims=True)
    e2 = jnp.min(jnp.where(probsm >= p2, ei, E), axis=-1, keepdims=True)
    mask2 = ei == e2

    ti = lax.broadcasted_iota(jnp.int32, (T, T), 0)
    tj = lax.broadcasted_iota(jnp.int32, (T, T), 1)
    L = (ti >= tj).astype(jnp.bfloat16)
    m1f = mask1.astype(jnp.bfloat16)
    m2f = mask2.astype(jnp.bfloat16)
    c1 = lax.dot_general(L, m1f, (((1,), (0,)), ((), ())),
                         preferred_element_type=jnp.float32)
    c2 = lax.dot_general(L, m2f, (((1,), (0,)), ((), ())),
                         preferred_element_type=jnp.float32)
    cnt1 = jnp.sum(jnp.where(mask1, 1.0, 0.0), axis=0, keepdims=True)
    rank1 = c1 - 1.0
    rank2 = c2 - 1.0 + cnt1
    r1 = jnp.sum(jnp.where(mask1, rank1, 0.0), axis=-1, keepdims=True)
    r2 = jnp.sum(jnp.where(mask2, rank2, 0.0), axis=-1, keepdims=True)
    keep1 = r1 < cap
    keep2 = r2 < cap

    e1f = e1.astype(jnp.float32)
    e2f = e2.astype(jnp.float32)
    d1 = e1f * cap + r1
    d2 = e2f * cap + r2
    zero = jnp.zeros_like(d1)
    d1g = jnp.where(keep1, d1, zero).astype(jnp.int32)
    d2g = jnp.where(keep2, d2, zero).astype(jnp.int32)
    w1 = jnp.where(keep1, p1, 0.0)
    w2 = jnp.where(keep2, p2, 0.0)

    padi = jnp.zeros((T, 1), jnp.int32)
    idx_ref[...] = jnp.concatenate(
        [d1g, d2g, padi, padi, padi, padi, padi, padi], axis=1)
    w1_ref[...] = jnp.broadcast_to(w1, (T, 16))
    w2_ref[...] = jnp.broadcast_to(w2, (T, 16))

    tcol = lax.broadcasted_iota(jnp.int32, (T, 1), 0)
    v = T - tcol
    vhi = (v // 256).astype(jnp.float32)
    vlo = (v % 256).astype(jnp.float32)
    ciota = lax.broadcasted_iota(jnp.int32, (T, cap), 1)
    hit1 = ciota == r1.astype(jnp.int32)
    hit2 = ciota == r2.astype(jnp.int32)
    d1hi = jnp.where(hit1, vhi, 0.0).astype(jnp.bfloat16)
    d1lo = jnp.where(hit1, vlo, 0.0).astype(jnp.bfloat16)
    d2hi = jnp.where(hit2, vhi, 0.0).astype(jnp.bfloat16)
    d2lo = jnp.where(hit2, vlo, 0.0).astype(jnp.bfloat16)
    dn = (((0,), (0,)), ((), ()))
    shi = (lax.dot_general(m1f, d1hi, dn, preferred_element_type=jnp.float32) +
           lax.dot_general(m2f, d2hi, dn, preferred_element_type=jnp.float32))
    slo = (lax.dot_general(m1f, d1lo, dn, preferred_element_type=jnp.float32) +
           lax.dot_general(m2f, d2lo, dn, preferred_element_type=jnp.float32))
    src_ref[...] = (jnp.float32(T) - (256.0 * shi + slo)).astype(jnp.int32)

    tokext_ref[0:T, :] = x
    tokext_ref[T:, :] = jnp.zeros((tokext_ref.shape[0] - T, x.shape[1]),
                                  jnp.float32)


def _ffn_body(x_ref, wi_ref, wo_ref, o_ref):
    j = pl.program_id(1)
    x = x_ref[...]
    wi = wi_ref[0]
    wo = wo_ref[0]
    h = lax.dot_general(x, wi, (((1,), (0,)), ((), ())),
                        preferred_element_type=jnp.float32)
    h = 0.5 * h * (1.0 + lax.erf(h * 0.7071067811865476))
    acc = lax.dot_general(h, wo, (((1,), (0,)), ((), ())),
                          preferred_element_type=jnp.float32)

    @pl.when(j == 0)
    def _():
        o_ref[...] = acc

    @pl.when(j > 0)
    def _():
        o_ref[...] += acc


def _make_dispatch(S, T, H, nw):
    spw = S // nw
    chunk = 16
    nbuf = 3
    nch = spw // chunk
    mesh = plsc.VectorSubcoreMesh(core_axis_name="c", subcore_axis_name="s")

    @functools.partial(
        pl.kernel, mesh=mesh,
        out_type=jax.ShapeDtypeStruct((S, H), jnp.float32),
        scratch_types=[
            pltpu.VMEM((spw,), jnp.int32),
            pltpu.VMEM((nbuf, chunk, H), jnp.float32),
            pltpu.SemaphoreType.DMA,
            pltpu.SemaphoreType.DMA((nbuf,)),
            pltpu.SemaphoreType.DMA((nbuf,)),
        ],
    )
    def dispatch(tokens_hbm, src_hbm, out_hbm, idx_v, rows_v, isem, gsem, wsem):
        wid = lax.axis_index("s") * 2 + lax.axis_index("c")
        r = wid % 4
        r = jnp.where(r == 2, 3, jnp.where(r == 3, 2, r))
        blk = (wid - (wid % 4)) + r
        base = blk * spw
        ic = pltpu.async_copy(src_hbm.at[pl.ds(base, spw)], idx_v, isem)

        def start_gather(j):
            return pltpu.async_copy(
                tokens_hbm.at[idx_v.at[pl.ds(j * chunk, chunk)]],
                rows_v.at[j % nbuf], gsem.at[j % nbuf])

        g = [None] * nch
        w = [None] * nch
        ic.wait()
        for j in range(min(nbuf, nch)):
            g[j] = start_gather(j)
        for j in range(nch):
            g[j].wait()
            w[j] = pltpu.async_copy(
                rows_v.at[j % nbuf], out_hbm.at[pl.ds(base + j * chunk, chunk)],
                wsem.at[j % nbuf])
            if j + nbuf < nch:
                w[j].wait()
                g[j + nbuf] = start_gather(j + nbuf)
        for j in range(max(0, nch - nbuf), nch):
            w[j].wait()

    return dispatch


def _make_combine(S, T, H, nw):
    tpw = T // nw
    chunk = 8
    nch = tpw // chunk
    mesh = plsc.VectorSubcoreMesh(core_axis_name="c", subcore_axis_name="s")

    @functools.partial(
        pl.kernel, mesh=mesh,
        out_type=jax.ShapeDtypeStruct((T, H), jnp.float32),
        scratch_types=[
            pltpu.VMEM((tpw,), jnp.int32),
            pltpu.VMEM((tpw,), jnp.int32),
            pltpu.VMEM((tpw, 16), jnp.float32),
            pltpu.VMEM((tpw, 16), jnp.float32),
            pltpu.VMEM((2, chunk, H), jnp.float32),
            pltpu.VMEM((2, chunk, H), jnp.float32),
            pltpu.VMEM((2, chunk, H), jnp.float32),
            pltpu.SemaphoreType.DMA((4,)),
            pltpu.SemaphoreType.DMA((2,)),
            pltpu.SemaphoreType.DMA((2,)),
            pltpu.SemaphoreType.DMA((2,)),
        ],
    )
    def combine(eo_hbm, i1_hbm, i2_hbm, w1_hbm, w2_hbm, out_hbm,
                ia_v, ib_v, wv1, wv2, b1, b2, ob, usem, gasem, gbsem, wsem):
        wid = lax.axis_index("s") * 2 + lax.axis_index("c")
        base = wid * tpw
        u0 = pltpu.async_copy(i1_hbm.at[pl.ds(base, tpw)], ia_v, usem.at[0])
        u1 = pltpu.async_copy(i2_hbm.at[pl.ds(base, tpw)], ib_v, usem.at[1])
        u2 = pltpu.async_copy(w1_hbm.at[pl.ds(base, tpw)], wv1, usem.at[2])
        u3 = pltpu.async_copy(w2_hbm.at[pl.ds(base, tpw)], wv2, usem.at[3])
        u0.wait()
        u1.wait()
        u2.wait()
        u3.wait()

        def start(j):
            b = j % 2
            ga = pltpu.async_copy(
                eo_hbm.at[ia_v.at[pl.ds(j * chunk, chunk)]], b1.at[b],
                gasem.at[b])
            gb = pltpu.async_copy(
                eo_hbm.at[ib_v.at[pl.ds(j * chunk, chunk)]], b2.at[b],
                gbsem.at[b])
            return ga, gb

        ga = [None] * nch
        gb = [None] * nch
        wr = [None] * nch
        ga[0], gb[0] = start(0)
        if nch > 1:
            ga[1], gb[1] = start(1)
        for j in range(nch):
            b = j % 2
            ga[j].wait()
            gb[j].wait()
            if j >= 2:
                wr[j - 2].wait()
            for i in range(chunk):
                w1v = wv1[j * chunk + i]
                w2v = wv2[j * chunk + i]

                @pl.loop(0, H, step=64)
                def _(k, i=i, b=b, w1v=w1v, w2v=w2v):
                    for u in range(4):
                        sl = pl.ds(k + u * 16, 16)
                        ob[b, i, sl] = w1v * b1[b, i, sl] + w2v * b2[b, i, sl]

            wr[j] = pltpu.async_copy(
                ob.at[b], out_hbm.at[pl.ds(base + j * chunk, chunk)],
                wsem.at[b])
            if j + 2 < nch:
                ga[j + 2], gb[j + 2] = start(j + 2)
        if nch >= 2:
            wr[nch - 2].wait()
        wr[nch - 1].wait()

    return combine


def kernel(inputs, gate_weight, wi, wo):
    B, Tseq, H = inputs.shape
    T = B * Tseq
    E = gate_weight.shape[0]
    I = wi.shape[2]
    cap = _capacity(T, E)
    S = E * cap
    nw = 32

    tokens = inputs.reshape(T, H).astype(jnp.float32)

    idx, w1w, w2w, slot_src, tokens_ext = pl.pallas_call(
        functools.partial(_router_body, cap),
        out_shape=(jax.ShapeDtypeStruct((T, 8), jnp.int32),
                   jax.ShapeDtypeStruct((T, 16), jnp.float32),
                   jax.ShapeDtypeStruct((T, 16), jnp.float32),
                   jax.ShapeDtypeStruct((E, cap), jnp.int32),
                   jax.ShapeDtypeStruct((T + 8, H), jnp.float32)),
    )(tokens, gate_weight.astype(jnp.float32))

    slot_src = slot_src.reshape(S)
    dispatch = _make_dispatch(S, T, H, nw)(tokens_ext, slot_src)

    nj = 2
    ib = I // nj
    eo = pl.pallas_call(
        _ffn_body,
        grid=(E, nj),
        in_specs=[
            pl.BlockSpec((cap, H), lambda e, j: (e, 0)),
            pl.BlockSpec((1, H, ib), lambda e, j: (e, 0, j)),
            pl.BlockSpec((1, ib, H), lambda e, j: (e, j, 0)),
        ],
        out_specs=pl.BlockSpec((cap, H), lambda e, j: (e, 0)),
        out_shape=jax.ShapeDtypeStruct((S, H), jnp.float32),
        compiler_params=pltpu.CompilerParams(
            dimension_semantics=("arbitrary", "arbitrary")),
    )(dispatch, wi.astype(jnp.float32), wo.astype(jnp.float32))

    i1 = idx[:, 0]
    i2 = idx[:, 1]
    out = _make_combine(S, T, H, nw)(eo, i1, i2, w1w, w2w)
    return out.reshape(inputs.shape)


# speedup vs baseline: 1.0043x; 1.0043x over previous
import functools
import math

import jax
import jax.numpy as jnp
from jax import lax
from jax.experimental import pallas as pl
from jax.experimental.pallas import tpu as pltpu
from jax.experimental.pallas import tpu_sc as plsc

_NUM_EXPERTS = 8
_TOP_K = 2
_CAP_FACTOR = 1.25
_MIN_CAPACITY = 4


def _capacity(num_tokens, num_experts):
    cap = math.floor(_TOP_K * _CAP_FACTOR * num_tokens / num_experts)
    cap += cap % 2
    return max(cap, _MIN_CAPACITY)


def _router_body(cap, tok_ref, gate_ref, idx_ref, w1_ref, w2_ref, src_ref,
                 tokext_ref):
    T = tok_ref.shape[0]
    E = gate_ref.shape[0]
    x = tok_ref[...]
    g = gate_ref[...]
    logits = lax.dot_general(x, g, (((1,), (1,)), ((), ())),
                             preferred_element_type=jnp.float32)
    m = jnp.max(logits, axis=-1, keepdims=True)
    p = jnp.exp(logits - m)
    probs = p / jnp.sum(p, axis=-1, keepdims=True)

    ei = lax.broadcasted_iota(jnp.int32, (T, E), 1)
    p1 = jnp.max(probs, axis=-1, keepdims=True)
    e1 = jnp.min(jnp.where(probs >= p1, ei, E), axis=-1, keepdims=True)
    mask1 = ei == e1
    probsm = jnp.where(mask1, -1.0, probs)
    p2 = jnp.max(probsm, axis=-1, keepdims=True)
    e2 = jnp.min(jnp.where(probsm >= p2, ei, E), axis=-1, keepdims=True)
    mask2 = ei == e2

    ti = lax.broadcasted_iota(jnp.int32, (T, T), 0)
    tj = lax.broadcasted_iota(jnp.int32, (T, T), 1)
    L = (ti >= tj).astype(jnp.bfloat16)
    m1f = mask1.astype(jnp.bfloat16)
    m2f = mask2.astype(jnp.bfloat16)
    c1 = lax.dot_general(L, m1f, (((1,), (0,)), ((), ())),
                         preferred_element_type=jnp.float32)
    c2 = lax.dot_general(L, m2f, (((1,), (0,)), ((), ())),
                         preferred_element_type=jnp.float32)
    cnt1 = jnp.sum(jnp.where(mask1, 1.0, 0.0), axis=0, keepdims=True)
    rank1 = c1 - 1.0
    rank2 = c2 - 1.0 + cnt1
    r1 = jnp.sum(jnp.where(mask1, rank1, 0.0), axis=-1, keepdims=True)
    r2 = jnp.sum(jnp.where(mask2, rank2, 0.0), axis=-1, keepdims=True)
    keep1 = r1 < cap
    keep2 = r2 < cap

    e1f = e1.astype(jnp.float32)
    e2f = e2.astype(jnp.float32)
    d1 = e1f * cap + r1
    d2 = e2f * cap + r2
    zero = jnp.zeros_like(d1)
    d1g = jnp.where(keep1, d1, zero).astype(jnp.int32)
    d2g = jnp.where(keep2, d2, zero).astype(jnp.int32)
    w1 = jnp.where(keep1, p1, 0.0)
    w2 = jnp.where(keep2, p2, 0.0)

    padi = jnp.zeros((T, 1), jnp.int32)
    idx_ref[...] = jnp.concatenate(
        [d1g, d2g, padi, padi, padi, padi, padi, padi], axis=1)
    w1_ref[...] = jnp.broadcast_to(w1, (T, 16))
    w2_ref[...] = jnp.broadcast_to(w2, (T, 16))

    tcol = lax.broadcasted_iota(jnp.int32, (T, 1), 0)
    v = T - tcol
    vhi = (v // 256).astype(jnp.float32)
    vlo = (v % 256).astype(jnp.float32)
    ciota = lax.broadcasted_iota(jnp.int32, (T, cap), 1)
    hit1 = ciota == r1.astype(jnp.int32)
    hit2 = ciota == r2.astype(jnp.int32)
    d1hi = jnp.where(hit1, vhi, 0.0).astype(jnp.bfloat16)
    d1lo = jnp.where(hit1, vlo, 0.0).astype(jnp.bfloat16)
    d2hi = jnp.where(hit2, vhi, 0.0).astype(jnp.bfloat16)
    d2lo = jnp.where(hit2, vlo, 0.0).astype(jnp.bfloat16)
    dn = (((0,), (0,)), ((), ()))
    shi = (lax.dot_general(m1f, d1hi, dn, preferred_element_type=jnp.float32) +
           lax.dot_general(m2f, d2hi, dn, preferred_element_type=jnp.float32))
    slo = (lax.dot_general(m1f, d1lo, dn, preferred_element_type=jnp.float32) +
           lax.dot_general(m2f, d2lo, dn, preferred_element_type=jnp.float32))
    src_ref[...] = (jnp.float32(T) - (256.0 * shi + slo)).astype(jnp.int32)

    tokext_ref[0:T, :] = x
    tokext_ref[T:, :] = jnp.zeros((tokext_ref.shape[0] - T, x.shape[1]),
                                  jnp.float32)


def _ffn_body(x_ref, wi_ref, wo_ref, o_ref):
    j = pl.program_id(1)
    x = x_ref[...]
    wi = wi_ref[0]
    wo = wo_ref[0]
    h = lax.dot_general(x, wi, (((1,), (0,)), ((), ())),
                        preferred_element_type=jnp.float32)
    h = 0.5 * h * (1.0 + lax.erf(h * 0.7071067811865476))
    acc = lax.dot_general(h, wo, (((1,), (0,)), ((), ())),
                          preferred_element_type=jnp.float32)

    @pl.when(j == 0)
    def _():
        o_ref[...] = acc

    @pl.when(j > 0)
    def _():
        o_ref[...] += acc


def _make_dispatch(S, T, H, nw):
    spw = S // nw
    chunk = 16
    nbuf = 3
    nch = spw // chunk
    mesh = plsc.VectorSubcoreMesh(core_axis_name="c", subcore_axis_name="s")

    @functools.partial(
        pl.kernel, mesh=mesh,
        out_type=jax.ShapeDtypeStruct((S, H), jnp.float32),
        scratch_types=[
            pltpu.VMEM((spw,), jnp.int32),
            pltpu.VMEM((nbuf, chunk, H), jnp.float32),
            pltpu.SemaphoreType.DMA,
            pltpu.SemaphoreType.DMA((nbuf,)),
            pltpu.SemaphoreType.DMA((nbuf,)),
        ],
    )
    def dispatch(tokens_hbm, src_hbm, out_hbm, idx_v, rows_v, isem, gsem, wsem):
        wid = lax.axis_index("s") * 2 + lax.axis_index("c")
        base = wid * spw
        ic = pltpu.async_copy(src_hbm.at[pl.ds(base, spw)], idx_v, isem)

        def start_gather(j):
            return pltpu.async_copy(
                tokens_hbm.at[idx_v.at[pl.ds(j * chunk, chunk)]],
                rows_v.at[j % nbuf], gsem.at[j % nbuf])

        g = [None] * nch
        w = [None] * nch
        ic.wait()
        for j in range(min(nbuf, nch)):
            g[j] = start_gather(j)
        for j in range(nch):
            g[j].wait()
            w[j] = pltpu.async_copy(
                rows_v.at[j % nbuf], out_hbm.at[pl.ds(base + j * chunk, chunk)],
                wsem.at[j % nbuf])
            if j + nbuf < nch:
                w[j].wait()
                g[j + nbuf] = start_gather(j + nbuf)
        for j in range(max(0, nch - nbuf), nch):
            w[j].wait()

    return dispatch


def _make_combine(S, T, H, nw):
    tpw = T // nw
    chunk = 8
    nch = tpw // chunk
    mesh = plsc.VectorSubcoreMesh(core_axis_name="c", subcore_axis_name="s")

    @functools.partial(
        pl.kernel, mesh=mesh,
        out_type=jax.ShapeDtypeStruct((T, H), jnp.float32),
        scratch_types=[
            pltpu.VMEM((tpw,), jnp.int32),
            pltpu.VMEM((tpw,), jnp.int32),
            pltpu.VMEM((tpw, 16), jnp.float32),
            pltpu.VMEM((tpw, 16), jnp.float32),
            pltpu.VMEM((2, chunk, H), jnp.float32),
            pltpu.VMEM((2, chunk, H), jnp.float32),
            pltpu.VMEM((2, chunk, H), jnp.float32),
            pltpu.SemaphoreType.DMA((4,)),
            pltpu.SemaphoreType.DMA((2,)),
            pltpu.SemaphoreType.DMA((2,)),
            pltpu.SemaphoreType.DMA((2,)),
        ],
    )
    def combine(eo_hbm, i1_hbm, i2_hbm, w1_hbm, w2_hbm, out_hbm,
                ia_v, ib_v, wv1, wv2, b1, b2, ob, usem, gasem, gbsem, wsem):
        wid = lax.axis_index("s") * 2 + lax.axis_index("c")
        base = wid * tpw
        u0 = pltpu.async_copy(i1_hbm.at[pl.ds(base, tpw)], ia_v, usem.at[0])
        u1 = pltpu.async_copy(i2_hbm.at[pl.ds(base, tpw)], ib_v, usem.at[1])
        u2 = pltpu.async_copy(w1_hbm.at[pl.ds(base, tpw)], wv1, usem.at[2])
        u3 = pltpu.async_copy(w2_hbm.at[pl.ds(base, tpw)], wv2, usem.at[3])
        u0.wait()
        u1.wait()
        u2.wait()
        u3.wait()

        def start(j):
            b = j % 2
            ga = pltpu.async_copy(
                eo_hbm.at[ia_v.at[pl.ds(j * chunk, chunk)]], b1.at[b],
                gasem.at[b])
            gb = pltpu.async_copy(
                eo_hbm.at[ib_v.at[pl.ds(j * chunk, chunk)]], b2.at[b],
                gbsem.at[b])
            return ga, gb

        ga = [None] * nch
        gb = [None] * nch
        wr = [None] * nch
        ga[0], gb[0] = start(0)
        if nch > 1:
            ga[1], gb[1] = start(1)
        for j in range(nch):
            b = j % 2
            ga[j].wait()
            gb[j].wait()
            if j >= 2:
                wr[j - 2].wait()
            for i in range(chunk):
                w1v = wv1[j * chunk + i]
                w2v = wv2[j * chunk + i]

                @pl.loop(0, H, step=64)
                def _(k, i=i, b=b, w1v=w1v, w2v=w2v):
                    for u in range(4):
                        sl = pl.ds(k + u * 16, 16)
                        ob[b, i, sl] = w1v * b1[b, i, sl] + w2v * b2[b, i, sl]

            wr[j] = pltpu.async_copy(
                ob.at[b], out_hbm.at[pl.ds(base + j * chunk, chunk)],
                wsem.at[b])
            if j + 2 < nch:
                ga[j + 2], gb[j + 2] = start(j + 2)
        if nch >= 2:
            wr[nch - 2].wait()
        wr[nch - 1].wait()

    return combine


def kernel(inputs, gate_weight, wi, wo):
    B, Tseq, H = inputs.shape
    T = B * Tseq
    E = gate_weight.shape[0]
    I = wi.shape[2]
    cap = _capacity(T, E)
    S = E * cap
    nw = 32

    tokens = inputs.reshape(T, H).astype(jnp.float32)

    idx, w1w, w2w, slot_src, tokens_ext = pl.pallas_call(
        functools.partial(_router_body, cap),
        out_shape=(jax.ShapeDtypeStruct((T, 8), jnp.int32),
                   jax.ShapeDtypeStruct((T, 16), jnp.float32),
                   jax.ShapeDtypeStruct((T, 16), jnp.float32),
                   jax.ShapeDtypeStruct((E, cap), jnp.int32),
                   jax.ShapeDtypeStruct((T + 8, H), jnp.float32)),
    )(tokens, gate_weight.astype(jnp.float32))

    slot_src = slot_src.reshape(S)
    dispatch = _make_dispatch(S, T, H, nw)(tokens_ext, slot_src)

    nj = 2
    ib = I // nj
    eo = pl.pallas_call(
        _ffn_body,
        grid=(E, nj),
        in_specs=[
            pl.BlockSpec((cap, H), lambda e, j: (e, 0)),
            pl.BlockSpec((1, H, ib), lambda e, j: (e, 0, j)),
            pl.BlockSpec((1, ib, H), lambda e, j: (e, j, 0)),
        ],
        out_specs=pl.BlockSpec((cap, H), lambda e, j: (e, 0)),
        out_shape=jax.ShapeDtypeStruct((S, H), jnp.float32),
        compiler_params=pltpu.CompilerParams(
            dimension_semantics=("arbitrary", "arbitrary")),
    )(dispatch, wi.astype(jnp.float32), wo.astype(jnp.float32))

    i1 = idx[:, 0]
    i2 = idx[:, 1]
    out = _make_combine(S, T, H, nw)(eo, i1, i2, w1w, w2w)
    return out.reshape(inputs.shape)
